# Initial kernel scaffold; baseline (speedup 1.0000x reference)
import jax
import jax.numpy as jnp
from jax.experimental import pallas as pl


def kernel(x, edge_attr, mlp_W1, mlp_b1, mlp_g, mlp_beta, mlp_W2, mlp_b2, edge_W, edge_b, bn_g, bn_b, edge_index, batch):
    raise NotImplementedError("write your pallas kernel here")


# speedup vs baseline: 3.0037x; 3.0037x over previous
import functools

import jax
import jax.numpy as jnp
from jax import lax
from jax.experimental import pallas as pl
from jax.experimental.pallas import tpu as pltpu
from jax.experimental.pallas import tpu_sc as plsc

N = 10000
E = 320000
D = 128
ED = 16
L = 3
G = 200
BN_EPS = 1e-5

NC = 2
NS = 16
NW = NC * NS
EW = E // NW
C = 80
NCH = EW // C
N_PAD = 10240
ROWS_PER_SUB = N_PAD // NS
ZROWS = 128


def _edge_proj_body(a_ref, w_ref, b_ref, o_ref):
    o_ref[...] = (
        jnp.dot(a_ref[...], w_ref[...], preferred_element_type=jnp.float32)
        + b_ref[...]
    )


def _edge_proj(edge_attr, W, b):
    BE = 4000
    grid = (E // BE,)
    return pl.pallas_call(
        _edge_proj_body,
        grid=grid,
        in_specs=[
            pl.BlockSpec((BE, ED), lambda i: (i, 0)),
            pl.BlockSpec((ED, D), lambda i: (0, 0)),
            pl.BlockSpec((1, D), lambda i: (0, 0)),
        ],
        out_specs=pl.BlockSpec((BE, D), lambda i: (i, 0)),
        out_shape=jax.ShapeDtypeStruct((E, D), jnp.float32),
    )(edge_attr, W, b.reshape(1, D))


def _sc_agg_body(h_hbm, e_hbm, src_hbm, dst_hbm, out_hbm,
                 src_v, dst_v, e_v, rows_v, zbuf_v, agg_sh, gsem):
    cid = lax.axis_index("c")
    sid = lax.axis_index("s")
    wid = sid * NC + cid

    def _zrow(r, _):
        for cc in range(D // 16):
            zbuf_v[r, pl.ds(cc * 16, 16)] = jnp.zeros((16,), jnp.float32)
        return 0

    lax.fori_loop(0, ZROWS, _zrow, 0)
    slab0 = sid * ROWS_PER_SUB
    for k in range(ROWS_PER_SUB // ZROWS):
        pltpu.sync_copy(zbuf_v, agg_sh.at[pl.ds(slab0 + k * ZROWS, ZROWS)])
    plsc.subcore_barrier()

    base_w = wid * EW

    def _chunk(g, _):
        off = base_w + g * C
        pltpu.sync_copy(src_hbm.at[pl.ds(off, C)], src_v)
        pltpu.sync_copy(dst_hbm.at[pl.ds(off, C)], dst_v)
        gather = pltpu.async_copy(h_hbm.at[src_v], rows_v, gsem)
        pltpu.sync_copy(e_hbm.at[pl.ds(off, C)], e_v)
        gather.wait()

        def _row(r, _):
            for cc in range(D // 16):
                sl = pl.ds(cc * 16, 16)
                rows_v[r, sl] = jnp.maximum(
                    rows_v[r, sl] + e_v[r, sl], 0.0)
            return 0

        lax.fori_loop(0, C, _row, 0)
        pltpu.sync_copy(rows_v, agg_sh.at[dst_v], add=True)
        return 0

    lax.fori_loop(0, NCH, _chunk, 0)
    plsc.subcore_barrier()

    pltpu.sync_copy(agg_sh.at[pl.ds(slab0, ROWS_PER_SUB)],
                    out_hbm.at[cid, pl.ds(slab0, ROWS_PER_SUB)])


@functools.partial(
    pl.kernel,
    out_type=jax.ShapeDtypeStruct((NC, N_PAD, D), jnp.float32),
    mesh=plsc.VectorSubcoreMesh(core_axis_name="c", subcore_axis_name="s"),
    scratch_types=[
        pltpu.VMEM((C,), jnp.int32),
        pltpu.VMEM((C,), jnp.int32),
        pltpu.VMEM((C, D), jnp.float32),
        pltpu.VMEM((C, D), jnp.float32),
        pltpu.VMEM((ZROWS, D), jnp.float32),
        pltpu.VMEM_SHARED((N_PAD, D), jnp.float32),
        pltpu.SemaphoreType.DMA,
    ],
)
def _sc_agg(h_hbm, e_hbm, src_hbm, dst_hbm, out_hbm,
            src_v, dst_v, e_v, rows_v, zbuf_v, agg_sh, gsem):
    _sc_agg_body(h_hbm, e_hbm, src_hbm, dst_hbm, out_hbm,
                 src_v, dst_v, e_v, rows_v, zbuf_v, agg_sh, gsem)


def _mlp_body(h_ref, a0_ref, a1_ref, w1_ref, s1_ref, c1_ref,
              w2_ref, s2_ref, c2_ref, o_ref):
    h = h_ref[...]
    t = h + a0_ref[...] + a1_ref[...]
    u = jnp.maximum(
        jnp.dot(t, w1_ref[...], preferred_element_type=jnp.float32)
        * s1_ref[...] + c1_ref[...], 0.0)
    v = (jnp.dot(u, w2_ref[...], preferred_element_type=jnp.float32)
         * s2_ref[...] + c2_ref[...])
    o_ref[...] = jnp.maximum(v, 0.0) + h


def _mlp(h, a0, a1, W1, s1, c1, W2, s2, c2):
    BN = 2000
    grid = (N // BN,)
    vspec = pl.BlockSpec((1, D), lambda i: (0, 0))
    return pl.pallas_call(
        _mlp_body,
        grid=grid,
        in_specs=[
            pl.BlockSpec((BN, D), lambda i: (i, 0)),
            pl.BlockSpec((BN, D), lambda i: (i, 0)),
            pl.BlockSpec((BN, D), lambda i: (i, 0)),
            pl.BlockSpec((D, D), lambda i: (0, 0)),
            vspec, vspec,
            pl.BlockSpec((D, D), lambda i: (0, 0)),
            vspec, vspec,
        ],
        out_specs=pl.BlockSpec((BN, D), lambda i: (i, 0)),
        out_shape=jax.ShapeDtypeStruct((N, D), jnp.float32),
    )(h, a0, a1, W1, s1.reshape(1, D), c1.reshape(1, D),
      W2, s2.reshape(1, D), c2.reshape(1, D))


def _pool_body(h_ref, b_ref, o_ref, acc_ref, cnt_ref):
    i = pl.program_id(0)
    nblk = pl.num_programs(0)
    brow = b_ref[0]
    gids = lax.broadcasted_iota(jnp.int32, (G, brow.shape[-1]), 0)
    mask = (gids == brow).astype(jnp.float32)
    part = jnp.dot(mask, h_ref[...], preferred_element_type=jnp.float32)
    cpart = jnp.broadcast_to(jnp.sum(mask, axis=1, keepdims=True), (G, D))

    @pl.when(i == 0)
    def _init():
        acc_ref[...] = part
        cnt_ref[...] = cpart

    @pl.when(i > 0)
    def _acc():
        acc_ref[...] += part
        cnt_ref[...] += cpart

    @pl.when(i == nblk - 1)
    def _fin():
        o_ref[...] = acc_ref[...] / jnp.maximum(cnt_ref[...], 1.0)


def _pool(h, batch):
    BN = 2000
    grid = (N // BN,)
    batch3 = batch.reshape(N // BN, 1, BN)
    return pl.pallas_call(
        _pool_body,
        grid=grid,
        in_specs=[
            pl.BlockSpec((BN, D), lambda i: (i, 0)),
            pl.BlockSpec((1, 1, BN), lambda i: (i, 0, 0)),
        ],
        out_specs=pl.BlockSpec((G, D), lambda i: (0, 0)),
        out_shape=jax.ShapeDtypeStruct((G, D), jnp.float32),
        scratch_shapes=[
            pltpu.VMEM((G, D), jnp.float32),
            pltpu.VMEM((G, D), jnp.float32),
        ],
    )(h, batch3)


def kernel(x, edge_attr, mlp_W1, mlp_b1, mlp_g, mlp_beta, mlp_W2, mlp_b2,
           edge_W, edge_b, bn_g, bn_b, edge_index, batch):
    src = edge_index[0]
    dst = edge_index[1]
    inv = 1.0 / jnp.sqrt(1.0 + BN_EPS)
    s1 = mlp_g * inv
    c1 = mlp_b1 * s1 + mlp_beta

    h = x
    for i in range(L):
        e = _edge_proj(edge_attr, edge_W[i], edge_b[i])
        aggp = _sc_agg(h, e, src, dst)
        s2 = bn_g[i] * inv
        c2 = mlp_b2 * s2 + bn_b[i]
        h = _mlp(h, aggp[0, :N], aggp[1, :N], mlp_W1, s1, c1, mlp_W2, s2, c2)
    return _pool(h, batch)
